# baseline (speedup 1.0000x reference)
import functools

import jax
import jax.numpy as jnp
from jax import lax
from jax.experimental import pallas as pl
from jax.experimental.pallas import tpu as pltpu
from jax.experimental.pallas import tpu_sc as plsc

D_EMB = 64
CHUNK = 128


@functools.lru_cache(maxsize=None)
def _make(batch: int, num_nodes: int, d_emb: int):
    info = plsc.get_sparse_core_info()
    nc, ns = info.num_cores, info.num_subcores
    nw = nc * ns
    b_per_w = batch // nw
    n_chunks = b_per_w // CHUNK
    mesh = plsc.VectorSubcoreMesh(core_axis_name="c", subcore_axis_name="s")

    @functools.partial(
        pl.kernel,
        mesh=mesh,
        out_type=(
            jax.ShapeDtypeStruct((batch, d_emb), jnp.float32),
            jax.ShapeDtypeStruct((batch, d_emb), jnp.float32),
        ),
        scratch_types=[
            pltpu.VMEM((b_per_w,), jnp.int32),
            pltpu.VMEM((b_per_w, d_emb), jnp.float32),
            pltpu.VMEM((b_per_w, d_emb), jnp.float32),
            pltpu.SemaphoreType.DMA,
            pltpu.SemaphoreType.DMA,
            pltpu.SemaphoreType.DMA,
        ],
        compiler_params=pltpu.CompilerParams(use_tc_tiling_on_sc=False),
    )
    def k(ids_hbm, tgt_hbm, ctx_hbm, out_t, out_c, idx_v, rows_t, rows_c,
          sem_t, sem_c, sem_o):
        wid = lax.axis_index("s") * nc + lax.axis_index("c")
        base = wid * b_per_w
        pltpu.sync_copy(ids_hbm.at[pl.ds(base, b_per_w)], idx_v)
        gathers_t = []
        gathers_c = []
        for j in range(n_chunks):
            sl = pl.ds(j * CHUNK, CHUNK)
            gathers_t.append(
                pltpu.async_copy(tgt_hbm.at[idx_v.at[sl]], rows_t.at[sl], sem_t))
            gathers_c.append(
                pltpu.async_copy(ctx_hbm.at[idx_v.at[sl]], rows_c.at[sl], sem_c))
        for g in gathers_t:
            g.wait()
        out_w_t = pltpu.async_copy(rows_t, out_t.at[pl.ds(base, b_per_w)], sem_o)
        for g in gathers_c:
            g.wait()
        out_w_c = pltpu.async_copy(rows_c, out_c.at[pl.ds(base, b_per_w)], sem_o)
        out_w_t.wait()
        out_w_c.wait()

    return k


def kernel(ids, E_target, E_context):
    ids = ids.astype(jnp.int32)
    k = _make(ids.shape[0], E_target.shape[0], E_target.shape[1])
    return k(ids, E_target, E_context)


# speedup vs baseline: 1.5635x; 1.5635x over previous
import functools

import jax
import jax.numpy as jnp
from jax import lax
from jax.experimental import pallas as pl
from jax.experimental.pallas import tpu as pltpu
from jax.experimental.pallas import tpu_sc as plsc

LANES = 16


@functools.lru_cache(maxsize=None)
def _make(batch: int, num_nodes: int, d_emb: int):
    info = plsc.get_sparse_core_info()
    nc, ns = info.num_cores, info.num_subcores
    nw = nc * ns
    b_per_w = batch // nw
    mesh = plsc.VectorSubcoreMesh(core_axis_name="c", subcore_axis_name="s")

    @functools.partial(
        pl.kernel,
        mesh=mesh,
        out_type=(
            jax.ShapeDtypeStruct((batch, d_emb), jnp.float32),
            jax.ShapeDtypeStruct((batch, d_emb), jnp.float32),
        ),
        scratch_types=[
            pltpu.VMEM((b_per_w + LANES,), jnp.int32),
            pltpu.VMEM((b_per_w, d_emb), jnp.float32),
            pltpu.SemaphoreType.DMA,
        ],
        compiler_params=pltpu.CompilerParams(use_tc_tiling_on_sc=True),
    )
    def k(ids_hbm, tgt_hbm, ctx_hbm, out_t, out_c, idx_v, rows_v, sem_g):
        wid = lax.axis_index("s") * nc + lax.axis_index("c")
        base = wid * b_per_w
        pltpu.sync_copy(ids_hbm.at[pl.ds(base, b_per_w)],
                        idx_v.at[pl.ds(0, b_per_w)])
        out_slice = pl.ds(base, b_per_w)
        for tbl, out_hbm in ((tgt_hbm, out_t), (ctx_hbm, out_c)):

            def body(i, _):
                row = idx_v[pl.ds(i, LANES)][0]
                pltpu.async_copy(tbl.at[row], rows_v.at[i], sem_g)
                return 0

            lax.fori_loop(0, b_per_w, body, 0)
            pltpu.make_async_copy(
                out_hbm.at[out_slice], rows_v, sem_g).wait()
            pltpu.sync_copy(rows_v, out_hbm.at[out_slice])

    return k


def kernel(ids, E_target, E_context):
    ids = ids.astype(jnp.int32)
    n, d = E_target.shape
    k = _make(ids.shape[0], n, d)
    return k(ids, E_target, E_context)


# speedup vs baseline: 1.5666x; 1.0020x over previous
import functools

import jax
import jax.numpy as jnp
from jax import lax
from jax.experimental import pallas as pl
from jax.experimental.pallas import tpu as pltpu
from jax.experimental.pallas import tpu_sc as plsc

LANES = 16
NSEM = 8


@functools.lru_cache(maxsize=None)
def _make(batch: int, num_nodes: int, d_emb: int):
    info = plsc.get_sparse_core_info()
    nc, ns = info.num_cores, info.num_subcores
    nw = nc * ns
    b_per_w = batch // nw
    rows_per_sem = b_per_w // NSEM
    mesh = plsc.VectorSubcoreMesh(core_axis_name="c", subcore_axis_name="s")

    @functools.partial(
        pl.kernel,
        mesh=mesh,
        out_type=(
            jax.ShapeDtypeStruct((batch, d_emb), jnp.float32),
            jax.ShapeDtypeStruct((batch, d_emb), jnp.float32),
        ),
        scratch_types=[
            pltpu.VMEM((b_per_w + LANES,), jnp.int32),
            pltpu.VMEM((b_per_w, d_emb), jnp.float32),
        ] + [pltpu.SemaphoreType.DMA] * NSEM,
        compiler_params=pltpu.CompilerParams(use_tc_tiling_on_sc=True),
    )
    def k(ids_hbm, tgt_hbm, ctx_hbm, out_t, out_c, idx_v, rows_v, *sems):
        wid = lax.axis_index("s") * nc + lax.axis_index("c")
        base = wid * b_per_w
        pltpu.sync_copy(ids_hbm.at[pl.ds(base, b_per_w)],
                        idx_v.at[pl.ds(0, b_per_w)])
        out_slice = pl.ds(base, b_per_w)
        for tbl, out_hbm in ((tgt_hbm, out_t), (ctx_hbm, out_c)):

            def body(i, _):
                for q in range(NSEM):
                    row = idx_v[pl.ds(i * NSEM + q, LANES)][0]
                    pltpu.async_copy(tbl.at[row], rows_v.at[i * NSEM + q],
                                     sems[q])
                return 0

            lax.fori_loop(0, rows_per_sem, body, 0)
            for q in range(NSEM):
                pltpu.make_async_copy(
                    out_hbm.at[pl.ds(base, rows_per_sem)],
                    rows_v.at[pl.ds(0, rows_per_sem)], sems[q]).wait()
            pltpu.sync_copy(rows_v, out_hbm.at[out_slice])

    return k


def kernel(ids, E_target, E_context):
    ids = ids.astype(jnp.int32)
    n, d = E_target.shape
    k = _make(ids.shape[0], n, d)
    return k(ids, E_target, E_context)
